# Initial kernel scaffold; baseline (speedup 1.0000x reference)
import jax
import jax.numpy as jnp
from jax.experimental import pallas as pl


def kernel(vi, atom_mol_batch, W1, b1, g1, be1, W2, b2, g2, be2, W3, b3):
    raise NotImplementedError("write your pallas kernel here")


# speedup vs baseline: 4.1511x; 4.1511x over previous
import jax
import jax.numpy as jnp
from jax import lax
from jax.experimental import pallas as pl
from jax.experimental.pallas import tpu as pltpu
from jax.experimental.pallas import tpu_sc as plsc

N = 320000
D = 128
M = 4096

NC = 2
NS = 16
NW = NC * NS
ROWS_PER_TILE = N // NW
BLK = 80
NBLK = ROWS_PER_TILE // BLK
STRIPE = M // NS


def _sc_body(vi_hbm, idx_hbm, sums_hbm, counts_hbm,
             row_buf, idx_buf, zrow_buf, ones_buf, acc_shared, cnt_shared,
             dma_sem):
  core = lax.axis_index("c")
  sub = lax.axis_index("s")
  base = (core * NS + sub) * ROWS_PER_TILE

  zeros16 = jnp.zeros((16,), jnp.float32)
  ones16 = jnp.ones((16,), jnp.float32)

  def _zero_row(i, _):
    for j in range(D // 16):
      zrow_buf[i, pl.ds(j * 16, 16)] = zeros16
    return 0
  lax.fori_loop(0, STRIPE, _zero_row, 0)

  def _fill_ones(i, _):
    for j in range(D // 16):
      ones_buf[i, pl.ds(j * 16, 16)] = ones16
    return 0
  lax.fori_loop(0, BLK, _fill_ones, 0)

  pltpu.sync_copy(zrow_buf, acc_shared.at[pl.ds(sub * STRIPE, STRIPE)])
  pltpu.sync_copy(zrow_buf, cnt_shared.at[pl.ds(sub * STRIPE, STRIPE)])
  plsc.subcore_barrier()

  def _block(b, _):
    off = base + b * BLK
    pltpu.sync_copy(idx_hbm.at[pl.ds(off, BLK)], idx_buf)
    pltpu.sync_copy(vi_hbm.at[pl.ds(off, BLK)], row_buf)
    pltpu.async_copy(row_buf, acc_shared.at[idx_buf], dma_sem,
                     add=True).wait()
    pltpu.async_copy(ones_buf, cnt_shared.at[idx_buf], dma_sem,
                     add=True).wait()
    return 0
  lax.fori_loop(0, NBLK, _block, 0)

  plsc.subcore_barrier()
  pltpu.sync_copy(acc_shared.at[pl.ds(sub * STRIPE, STRIPE)], zrow_buf)
  pltpu.sync_copy(
      zrow_buf, sums_hbm.at[pl.ds(core * M + sub * STRIPE, STRIPE)])
  pltpu.sync_copy(cnt_shared.at[pl.ds(sub * STRIPE, STRIPE)], zrow_buf)
  pltpu.sync_copy(
      zrow_buf, counts_hbm.at[pl.ds(core * M + sub * STRIPE, STRIPE)])


@jax.jit
def _sc_segsum(vi, idx):
  mesh = plsc.VectorSubcoreMesh(
      core_axis_name="c", subcore_axis_name="s", num_cores=NC,
      num_subcores=NS)
  f = pl.kernel(
      _sc_body,
      out_type=(
          jax.ShapeDtypeStruct((NC * M, D), jnp.float32),
          jax.ShapeDtypeStruct((NC * M, D), jnp.float32),
      ),
      mesh=mesh,
      scratch_types=[
          pltpu.VMEM((BLK, D), jnp.float32),
          pltpu.VMEM((BLK,), jnp.int32),
          pltpu.VMEM((STRIPE, D), jnp.float32),
          pltpu.VMEM((BLK, D), jnp.float32),
          pltpu.VMEM_SHARED((M, D), jnp.float32),
          pltpu.VMEM_SHARED((M, D), jnp.float32),
          pltpu.SemaphoreType.DMA,
      ],
  )
  return f(vi, idx)


def _tc_body(parts_ref, counts_ref, w1_ref, b1_ref, g1_ref, be1_ref,
             w2_ref, b2_ref, g2_ref, be2_ref, w3_ref, b3_ref, out_ref):
  seg = parts_ref[0:M] + parts_ref[M:2 * M]
  cnt = counts_ref[0:M, 0:1] + counts_ref[M:2 * M, 0:1]
  mean = seg / jnp.maximum(cnt, 1.0)

  h = jnp.dot(mean, w1_ref[...], preferred_element_type=jnp.float32)
  h = h + b1_ref[...]
  mu = jnp.mean(h, axis=0, keepdims=True)
  var = jnp.mean((h - mu) * (h - mu), axis=0, keepdims=True)
  h = (h - mu) / jnp.sqrt(var + 1e-5) * g1_ref[...] + be1_ref[...]
  h = jnp.maximum(h, 0.0)

  h = jnp.dot(h, w2_ref[...], preferred_element_type=jnp.float32)
  h = h + b2_ref[...]
  mu = jnp.mean(h, axis=0, keepdims=True)
  var = jnp.mean((h - mu) * (h - mu), axis=0, keepdims=True)
  h = (h - mu) / jnp.sqrt(var + 1e-5) * g2_ref[...] + be2_ref[...]
  h = jnp.maximum(h, 0.0)

  out_ref[...] = (
      jnp.dot(h, w3_ref[...], preferred_element_type=jnp.float32)
      + b3_ref[...])


@jax.jit
def _tc_mlp(parts, counts, W1, b1, g1, be1, W2, b2, g2, be2, W3, b3):
  return pl.pallas_call(
      _tc_body,
      out_shape=jax.ShapeDtypeStruct((M, 1), jnp.float32),
  )(parts, counts, W1, b1, g1, be1, W2, b2, g2, be2, W3, b3)


def kernel(vi, atom_mol_batch, W1, b1, g1, be1, W2, b2, g2, be2, W3, b3):
  idx = atom_mol_batch.astype(jnp.int32)
  parts, counts = _sc_segsum(vi, idx)
  return _tc_mlp(
      parts, counts, W1, b1.reshape(1, D), g1.reshape(1, D),
      be1.reshape(1, D), W2, b2.reshape(1, D), g2.reshape(1, D),
      be2.reshape(1, D), W3, b3.reshape(1, 1))


# speedup vs baseline: 6.9681x; 1.6786x over previous
import jax
import jax.numpy as jnp
from jax import lax
from jax.experimental import pallas as pl
from jax.experimental.pallas import tpu as pltpu
from jax.experimental.pallas import tpu_sc as plsc

N = 320000
D = 128
M = 4096

NC = 2
NS = 16
NW = NC * NS
ROWS_PER_TILE = N // NW
BLK = 80
NBLK = ROWS_PER_TILE // BLK
STRIPE = M // NS
CH = 64


def _sc_body(vi_hbm, idx_hbm, sums_hbm, counts_hbm,
             buf_a, buf_b, idx_all, ones_buf, acc_shared, cnt_shared,
             gsem_a, gsem_b, ssem):
  core = lax.axis_index("c")
  sub = lax.axis_index("s")
  wid = core * NS + sub
  base = wid * ROWS_PER_TILE

  zeros16 = jnp.zeros((16,), jnp.float32)
  ones16 = jnp.ones((16,), jnp.float32)

  def _zero_row(i, _):
    for j in range(D // 16):
      buf_a[i, pl.ds(j * 16, 16)] = zeros16
    return 0
  lax.fori_loop(0, BLK, _zero_row, 0)

  def _fill_ones(i, _):
    for j in range(D // 16):
      ones_buf[i, pl.ds(j * 16, 16)] = ones16
    return 0
  lax.fori_loop(0, BLK, _fill_ones, 0)

  zsl = buf_a.at[pl.ds(0, CH)]
  for c in range(STRIPE // CH):
    pltpu.sync_copy(zsl, acc_shared.at[pl.ds(sub * STRIPE + c * CH, CH)])
    pltpu.sync_copy(zsl, cnt_shared.at[pl.ds(sub * STRIPE + c * CH, CH)])

  pltpu.sync_copy(idx_hbm.at[wid], idx_all)

  pltpu.async_copy(vi_hbm.at[pl.ds(base, BLK)], buf_a, gsem_a)
  pltpu.async_copy(vi_hbm.at[pl.ds(base + BLK, BLK)], buf_b, gsem_b)
  plsc.subcore_barrier()

  def _half(s, buf, gsem):
    pltpu.make_async_copy(
        vi_hbm.at[pl.ds(base + s * BLK, BLK)], buf, gsem).wait()
    iv = idx_all.at[s]
    cp1 = pltpu.async_copy(buf, acc_shared.at[iv], ssem, add=True)
    cp2 = pltpu.async_copy(ones_buf, cnt_shared.at[iv], ssem, add=True)
    cp1.wait()
    cp2.wait()
    @pl.when(s + 2 < NBLK)
    def _():
      pltpu.async_copy(vi_hbm.at[pl.ds(base + (s + 2) * BLK, BLK)], buf,
                       gsem)

  def _pair(t, _):
    _half(2 * t, buf_a, gsem_a)
    _half(2 * t + 1, buf_b, gsem_b)
    return 0
  lax.fori_loop(0, (NBLK - 1) // 2, _pair, 0)
  _half(NBLK - 1, buf_a, gsem_a)

  plsc.subcore_barrier()
  asl = buf_a.at[pl.ds(0, CH)]
  csl = buf_b.at[pl.ds(0, CH)]
  for c in range(STRIPE // CH):
    srow = sub * STRIPE + c * CH
    drow = core * M + srow
    pltpu.sync_copy(acc_shared.at[pl.ds(srow, CH)], asl)
    pltpu.sync_copy(asl, sums_hbm.at[pl.ds(drow, CH)])
    pltpu.sync_copy(cnt_shared.at[pl.ds(srow, CH)], csl)
    pltpu.sync_copy(csl, counts_hbm.at[pl.ds(drow, CH)])


@jax.jit
def _sc_segsum(vi, idx):
  mesh = plsc.VectorSubcoreMesh(
      core_axis_name="c", subcore_axis_name="s", num_cores=NC,
      num_subcores=NS)
  f = pl.kernel(
      _sc_body,
      out_type=(
          jax.ShapeDtypeStruct((NC * M, D), jnp.float32),
          jax.ShapeDtypeStruct((NC * M, D), jnp.float32),
      ),
      mesh=mesh,
      scratch_types=[
          pltpu.VMEM((BLK, D), jnp.float32),
          pltpu.VMEM((BLK, D), jnp.float32),
          pltpu.VMEM((NBLK, BLK), jnp.int32),
          pltpu.VMEM((BLK, D), jnp.float32),
          pltpu.VMEM_SHARED((M, D), jnp.float32),
          pltpu.VMEM_SHARED((M, D), jnp.float32),
          pltpu.SemaphoreType.DMA,
          pltpu.SemaphoreType.DMA,
          pltpu.SemaphoreType.DMA,
      ],
  )
  return f(vi, idx.reshape(NW, NBLK, BLK))


def _tc_body(parts_ref, counts_ref, w1_ref, b1_ref, g1_ref, be1_ref,
             w2_ref, b2_ref, g2_ref, be2_ref, w3_ref, b3_ref, out_ref):
  seg = parts_ref[0:M] + parts_ref[M:2 * M]
  cnt = counts_ref[0:M, 0:1] + counts_ref[M:2 * M, 0:1]
  mean = seg / jnp.maximum(cnt, 1.0)

  h = jnp.dot(mean, w1_ref[...], preferred_element_type=jnp.float32)
  h = h + b1_ref[...]
  mu = jnp.mean(h, axis=0, keepdims=True)
  var = jnp.mean((h - mu) * (h - mu), axis=0, keepdims=True)
  h = (h - mu) / jnp.sqrt(var + 1e-5) * g1_ref[...] + be1_ref[...]
  h = jnp.maximum(h, 0.0)

  h = jnp.dot(h, w2_ref[...], preferred_element_type=jnp.float32)
  h = h + b2_ref[...]
  mu = jnp.mean(h, axis=0, keepdims=True)
  var = jnp.mean((h - mu) * (h - mu), axis=0, keepdims=True)
  h = (h - mu) / jnp.sqrt(var + 1e-5) * g2_ref[...] + be2_ref[...]
  h = jnp.maximum(h, 0.0)

  out_ref[...] = (
      jnp.dot(h, w3_ref[...], preferred_element_type=jnp.float32)
      + b3_ref[...])


@jax.jit
def _tc_mlp(parts, counts, W1, b1, g1, be1, W2, b2, g2, be2, W3, b3):
  return pl.pallas_call(
      _tc_body,
      out_shape=jax.ShapeDtypeStruct((M, 1), jnp.float32),
  )(parts, counts, W1, b1, g1, be1, W2, b2, g2, be2, W3, b3)


def kernel(vi, atom_mol_batch, W1, b1, g1, be1, W2, b2, g2, be2, W3, b3):
  idx = atom_mol_batch.astype(jnp.int32)
  parts, counts = _sc_segsum(vi, idx)
  return _tc_mlp(
      parts, counts, W1, b1.reshape(1, D), g1.reshape(1, D),
      be1.reshape(1, D), W2, b2.reshape(1, D), g2.reshape(1, D),
      be2.reshape(1, D), W3, b3.reshape(1, 1))
